# baseline (speedup 1.0000x reference)
import jax
import jax.numpy as jnp
from jax import lax
from jax.experimental import pallas as pl
from jax.experimental.pallas import tpu as pltpu
from jax.experimental.pallas import tpu_sc as plsc

_VOC = 1000
_EMB = 64
_C = 4
_K = 20
_NIDX = _C + 1 + _K
_NSC = _K + 1
_KPAD = 24
_NW = 32
_L = 16


def _sc_scores(idx_hbm, u_hbm, out_hbm, u_v, idx_v, s_v):
    bpw = idx_v.shape[1]
    cid = lax.axis_index("c")
    sid = lax.axis_index("s")
    w = sid * 2 + cid

    pltpu.sync_copy(u_hbm, u_v)
    pltpu.sync_copy(idx_hbm.at[w], idx_v)

    zeros = jnp.zeros((_L,), jnp.float32)

    def g_body(g, carry):
        b0 = g * _L
        bases = [idx_v[k, pl.ds(b0, _L)] * _EMB for k in range(_NIDX)]

        def e_body(e, accs):
            ctx = [plsc.load_gather(u_v, [bases[c] + e]) for c in range(_C)]
            h = ((ctx[0] + ctx[1]) + (ctx[2] + ctx[3])) * (1.0 / _C)
            return tuple(
                accs[t] + plsc.load_gather(u_v, [bases[_C + t] + e]) * h
                for t in range(_NSC))

        accs = lax.fori_loop(0, _EMB, e_body, (zeros,) * _NSC)
        for t in range(_NSC):
            s_v[t, pl.ds(b0, _L)] = accs[t]
        for t in range(_NSC, _KPAD):
            s_v[t, pl.ds(b0, _L)] = zeros
        return carry

    lax.fori_loop(0, bpw // _L, g_body, 0)
    pltpu.sync_copy(s_v, out_hbm.at[w])


def _logsig_reduce(s_ref, o_ref):
    s = s_ref[...]
    row = lax.broadcasted_iota(jnp.int32, s.shape, 0)
    kk = row % _KPAD
    sp = jnp.maximum(s, 0.0) + jnp.log1p(jnp.exp(-jnp.abs(s)))
    contrib = jnp.where(kk >= _NSC, 0.0,
                        jnp.where(kk == 0, s, 0.0) - sp)
    o_ref[0, 0] = -jnp.sum(contrib)


def kernel(x, target, neg_samples, U):
    B = x.shape[0]
    bpw = B // _NW

    idx_all = jnp.concatenate(
        [x.T, target[None, :], neg_samples.T], axis=0).astype(jnp.int32)
    idx_blk = idx_all.reshape(_NIDX, _NW, bpw).transpose(1, 0, 2)
    u_flat = U.reshape(-1)

    mesh = plsc.VectorSubcoreMesh(core_axis_name="c", subcore_axis_name="s")
    scores = pl.kernel(
        _sc_scores,
        out_type=jax.ShapeDtypeStruct((_NW, _KPAD, bpw), jnp.float32),
        mesh=mesh,
        scratch_types=[
            pltpu.VMEM((_VOC * _EMB,), jnp.float32),
            pltpu.VMEM((_NIDX, bpw), jnp.int32),
            pltpu.VMEM((_KPAD, bpw), jnp.float32),
        ],
        compiler_params=pltpu.CompilerParams(needs_layout_passes=False),
    )(idx_blk, u_flat)

    loss = pl.pallas_call(
        _logsig_reduce,
        grid=(1,),
        in_specs=[pl.BlockSpec((_NW * _KPAD, bpw), lambda i: (0, 0))],
        out_specs=pl.BlockSpec(memory_space=pltpu.SMEM),
        out_shape=jax.ShapeDtypeStruct((1, 1), jnp.float32),
    )(scores.reshape(_NW * _KPAD, bpw))
    return loss[0, 0]


# speedup vs baseline: 1.1878x; 1.1878x over previous
import jax
import jax.numpy as jnp
from jax import lax
from jax.experimental import pallas as pl
from jax.experimental.pallas import tpu as pltpu
from jax.experimental.pallas import tpu_sc as plsc

_VOC = 1000
_EMB = 64
_C = 4
_K = 20
_NIDX = _C + 1 + _K
_NSC = _K + 1
_KPAD = 24
_NW = 32
_L = 16
_EB = 16


def _sc_scores(idx_hbm, u_hbm, out_hbm, u_v, idx_v, s_v):
    bpw = idx_v.shape[1]
    cid = lax.axis_index("c")
    sid = lax.axis_index("s")
    w = sid * 2 + cid

    pltpu.sync_copy(u_hbm, u_v)
    pltpu.sync_copy(idx_hbm.at[w], idx_v)

    zeros = jnp.zeros((_L,), jnp.float32)

    def pad_zero(i, carry):
        for t in range(_NSC, _KPAD):
            s_v[t, pl.ds(i * _L, _L)] = zeros
        return carry

    lax.fori_loop(0, bpw // _L, pad_zero, 0)

    def g_body(g, carry):
        b0 = g * _L
        for t in range(_NSC):
            s_v[t, pl.ds(b0, _L)] = zeros

        ctx_base = [idx_v[c, pl.ds(b0, _L)] * _EMB for c in range(_C)]

        def eb_body(eb, carry):
            e0 = eb * _EB
            cb = [b + e0 for b in ctx_base]
            h = []
            for j in range(_EB):
                g0 = plsc.load_gather(u_v, [cb[0] + j])
                g1 = plsc.load_gather(u_v, [cb[1] + j])
                g2 = plsc.load_gather(u_v, [cb[2] + j])
                g3 = plsc.load_gather(u_v, [cb[3] + j])
                h.append(((g0 + g1) + (g2 + g3)) * (1.0 / _C))

            for t in range(_NSC):
                bte = idx_v[_C + t, pl.ds(b0, _L)] * _EMB + e0
                p = [zeros, zeros, zeros, zeros]
                for j in range(_EB):
                    u = plsc.load_gather(u_v, [bte + j])
                    p[j % 4] = p[j % 4] + u * h[j]
                acc = s_v[t, pl.ds(b0, _L)]
                s_v[t, pl.ds(b0, _L)] = (
                    acc + ((p[0] + p[1]) + (p[2] + p[3])))
            return carry

        lax.fori_loop(0, _EMB // _EB, eb_body, 0)
        return carry

    lax.fori_loop(0, bpw // _L, g_body, 0)
    pltpu.sync_copy(s_v, out_hbm.at[w])


def _logsig_reduce(s_ref, o_ref):
    s = s_ref[...]
    row = lax.broadcasted_iota(jnp.int32, s.shape, 0)
    kk = row % _KPAD
    sp = jnp.maximum(s, 0.0) + jnp.log1p(jnp.exp(-jnp.abs(s)))
    contrib = jnp.where(kk >= _NSC, 0.0,
                        jnp.where(kk == 0, s, 0.0) - sp)
    o_ref[0, 0] = -jnp.sum(contrib)


def kernel(x, target, neg_samples, U):
    B = x.shape[0]
    bpw = B // _NW

    idx_all = jnp.concatenate(
        [x.T, target[None, :], neg_samples.T], axis=0).astype(jnp.int32)
    idx_blk = idx_all.reshape(_NIDX, _NW, bpw).transpose(1, 0, 2)
    u_flat = U.reshape(-1)

    mesh = plsc.VectorSubcoreMesh(core_axis_name="c", subcore_axis_name="s")
    scores = pl.kernel(
        _sc_scores,
        out_type=jax.ShapeDtypeStruct((_NW, _KPAD, bpw), jnp.float32),
        mesh=mesh,
        scratch_types=[
            pltpu.VMEM((_VOC * _EMB,), jnp.float32),
            pltpu.VMEM((_NIDX, bpw), jnp.int32),
            pltpu.VMEM((_KPAD, bpw), jnp.float32),
        ],
        compiler_params=pltpu.CompilerParams(needs_layout_passes=False),
    )(idx_blk, u_flat)

    loss = pl.pallas_call(
        _logsig_reduce,
        grid=(1,),
        in_specs=[pl.BlockSpec((_NW * _KPAD, bpw), lambda i: (0, 0))],
        out_specs=pl.BlockSpec(memory_space=pltpu.SMEM),
        out_shape=jax.ShapeDtypeStruct((1, 1), jnp.float32),
    )(scores.reshape(_NW * _KPAD, bpw))
    return loss[0, 0]


# speedup vs baseline: 4.2332x; 3.5640x over previous
import jax
import jax.numpy as jnp
from jax import lax
from jax.experimental import pallas as pl
from jax.experimental.pallas import tpu as pltpu
from jax.experimental.pallas import tpu_sc as plsc

_VOC = 1000
_EMB = 64
_C = 4
_K = 20
_NIDX = _C + 1 + _K
_NSC = _K + 1
_KPAD = 24
_NW = 32
_L = 16
_EB = 16
_STR = 65


def _sc_scores(idx_hbm, u_hbm, out_hbm, u_v, idx_v, s_v):
    bpw = idx_v.shape[1]
    cid = lax.axis_index("c")
    sid = lax.axis_index("s")
    w = sid * 2 + cid

    pltpu.sync_copy(u_hbm, u_v)
    pltpu.sync_copy(idx_hbm.at[w], idx_v)

    zeros = jnp.zeros((_L,), jnp.float32)

    def pad_zero(i, carry):
        for t in range(_NSC, _KPAD):
            s_v[t, pl.ds(i * _L, _L)] = zeros
        return carry

    lax.fori_loop(0, bpw // _L, pad_zero, 0)

    def g_body(g, carry):
        b0 = g * _L
        for t in range(_NSC):
            s_v[t, pl.ds(b0, _L)] = zeros

        ctx_base = [idx_v[c, pl.ds(b0, _L)] * _STR for c in range(_C)]

        def eb_body(eb, carry):
            e0 = eb * _EB
            cb = [b + e0 for b in ctx_base]
            h = []
            for j in range(_EB):
                g0 = plsc.load_gather(u_v, [cb[0] + j])
                g1 = plsc.load_gather(u_v, [cb[1] + j])
                g2 = plsc.load_gather(u_v, [cb[2] + j])
                g3 = plsc.load_gather(u_v, [cb[3] + j])
                h.append(((g0 + g1) + (g2 + g3)) * (1.0 / _C))

            for t in range(_NSC):
                bte = idx_v[_C + t, pl.ds(b0, _L)] * _STR + e0
                p = [zeros, zeros, zeros, zeros]
                for j in range(_EB):
                    u = plsc.load_gather(u_v, [bte + j])
                    p[j % 4] = p[j % 4] + u * h[j]
                acc = s_v[t, pl.ds(b0, _L)]
                s_v[t, pl.ds(b0, _L)] = (
                    acc + ((p[0] + p[1]) + (p[2] + p[3])))
            return carry

        lax.fori_loop(0, _EMB // _EB, eb_body, 0)
        return carry

    lax.fori_loop(0, bpw // _L, g_body, 0)
    pltpu.sync_copy(s_v, out_hbm.at[w])


def _logsig_reduce(s_ref, o_ref):
    s = s_ref[...]
    row = lax.broadcasted_iota(jnp.int32, s.shape, 0)
    kk = row % _KPAD
    sp = jnp.maximum(s, 0.0) + jnp.log1p(jnp.exp(-jnp.abs(s)))
    contrib = jnp.where(kk >= _NSC, 0.0,
                        jnp.where(kk == 0, s, 0.0) - sp)
    o_ref[0, 0] = -jnp.sum(contrib)


def kernel(x, target, neg_samples, U):
    B = x.shape[0]
    bpw = B // _NW

    idx_all = jnp.concatenate(
        [x.T, target[None, :], neg_samples.T], axis=0).astype(jnp.int32)
    idx_blk = idx_all.reshape(_NIDX, _NW, bpw).transpose(1, 0, 2)
    u_flat = jnp.pad(U, ((0, 0), (0, _STR - _EMB))).reshape(-1)

    mesh = plsc.VectorSubcoreMesh(core_axis_name="c", subcore_axis_name="s")
    scores = pl.kernel(
        _sc_scores,
        out_type=jax.ShapeDtypeStruct((_NW, _KPAD, bpw), jnp.float32),
        mesh=mesh,
        scratch_types=[
            pltpu.VMEM((_VOC * _STR,), jnp.float32),
            pltpu.VMEM((_NIDX, bpw), jnp.int32),
            pltpu.VMEM((_KPAD, bpw), jnp.float32),
        ],
        compiler_params=pltpu.CompilerParams(needs_layout_passes=False),
    )(idx_blk, u_flat)

    loss = pl.pallas_call(
        _logsig_reduce,
        grid=(1,),
        in_specs=[pl.BlockSpec((_NW * _KPAD, bpw), lambda i: (0, 0))],
        out_specs=pl.BlockSpec(memory_space=pltpu.SMEM),
        out_shape=jax.ShapeDtypeStruct((1, 1), jnp.float32),
    )(scores.reshape(_NW * _KPAD, bpw))
    return loss[0, 0]


# speedup vs baseline: 5.3692x; 1.2684x over previous
import jax
import jax.numpy as jnp
from jax import lax
from jax.experimental import pallas as pl
from jax.experimental.pallas import tpu as pltpu
from jax.experimental.pallas import tpu_sc as plsc

_VOC = 1000
_EMB = 64
_C = 4
_K = 20
_NIDX = _C + 1 + _K
_NSC = _K + 1
_KPAD = 24
_NW = 32
_L = 16
_GCH = 128
_FK = 12


def _tc_gram(u_ref, g_ref):
    U = u_ref[...]
    g_ref[...] = lax.dot_general(
        U, U, (((1,), (1,)), ((), ())),
        preferred_element_type=jnp.float32,
        precision=jax.lax.Precision.HIGHEST)


def _sc_vals(idx_hbm, g_hbm, out_hbm, g_sp, idx_v, lst_v, val_v, s_v, sem):
    bpw = idx_v.shape[1]
    cid = lax.axis_index("c")
    sid = lax.axis_index("s")
    w = sid * 2 + cid

    pltpu.sync_copy(idx_hbm.at[w], idx_v)

    @pl.when(sid == 0)
    def _():
        pltpu.sync_copy(g_hbm, g_sp)

    zeros = jnp.zeros((_L,), jnp.float32)

    def pad_body(g, carry):
        for t in range(_NSC, _KPAD):
            s_v[t, pl.ds(g * _L, _L)] = zeros
        return carry

    lax.fori_loop(0, bpw // _L, pad_body, 0)

    barriered = False
    for c in range(_C):
        def lst_body(g, carry):
            b0 = g * _L
            xc = idx_v[c, pl.ds(b0, _L)]
            for t in range(_NSC):
                tv = idx_v[_C + t, pl.ds(b0, _L)] * _VOC
                lst_v[pl.ds(t * bpw + b0, _L)] = tv + xc
            return carry

        lax.fori_loop(0, bpw // _L, lst_body, 0)

        if not barriered:
            plsc.subcore_barrier()
            barriered = True

        nch = _NSC * bpw // _GCH
        def fire_body(i, carry):
            descs = []
            for k in range(_FK):
                o = (i * _FK + k) * _GCH
                descs.append(pltpu.async_copy(
                    g_sp.at[lst_v.at[pl.ds(o, _GCH)]],
                    val_v.at[pl.ds(o, _GCH)], sem))
            for d in descs:
                d.wait()
            return carry

        lax.fori_loop(0, nch // _FK, fire_body, 0)

        def sum_body(g, carry):
            b0 = g * _L
            for t in range(_NSC):
                v = val_v[pl.ds(t * bpw + b0, _L)]
                if c == 0:
                    s_v[t, pl.ds(b0, _L)] = v
                elif c == _C - 1:
                    s_v[t, pl.ds(b0, _L)] = (
                        (s_v[t, pl.ds(b0, _L)] + v) * (1.0 / _C))
                else:
                    s_v[t, pl.ds(b0, _L)] = s_v[t, pl.ds(b0, _L)] + v
            return carry

        lax.fori_loop(0, bpw // _L, sum_body, 0)

    pltpu.sync_copy(s_v, out_hbm.at[w])


def _logsig_reduce(s_ref, o_ref):
    s = s_ref[...]
    row = lax.broadcasted_iota(jnp.int32, s.shape, 0)
    kk = row % _KPAD
    sp = jnp.maximum(s, 0.0) + jnp.log1p(jnp.exp(-jnp.abs(s)))
    contrib = jnp.where(kk >= _NSC, 0.0,
                        jnp.where(kk == 0, s, 0.0) - sp)
    o_ref[0, 0] = -jnp.sum(contrib)


def kernel(x, target, neg_samples, U):
    B = x.shape[0]
    bpw = B // _NW

    idx_all = jnp.concatenate(
        [x.T, target[None, :], neg_samples.T], axis=0).astype(jnp.int32)
    idx_blk = idx_all.reshape(_NIDX, _NW, bpw).transpose(1, 0, 2)

    gram = pl.pallas_call(
        _tc_gram,
        grid=(1,),
        in_specs=[pl.BlockSpec((_VOC, _EMB), lambda i: (0, 0))],
        out_specs=pl.BlockSpec((_VOC, _VOC), lambda i: (0, 0)),
        out_shape=jax.ShapeDtypeStruct((_VOC, _VOC), jnp.float32),
    )(U)

    mesh = plsc.VectorSubcoreMesh(core_axis_name="c", subcore_axis_name="s")
    scores = pl.kernel(
        _sc_vals,
        out_type=jax.ShapeDtypeStruct((_NW, _KPAD, bpw), jnp.float32),
        mesh=mesh,
        scratch_types=[
            pltpu.VMEM_SHARED((_VOC * _VOC,), jnp.float32),
            pltpu.VMEM((_NIDX, bpw), jnp.int32),
            pltpu.VMEM((_NSC * bpw,), jnp.int32),
            pltpu.VMEM((_NSC * bpw,), jnp.float32),
            pltpu.VMEM((_KPAD, bpw), jnp.float32),
            pltpu.SemaphoreType.DMA,
        ],
        compiler_params=pltpu.CompilerParams(needs_layout_passes=False),
    )(idx_blk, gram.reshape(-1))

    loss = pl.pallas_call(
        _logsig_reduce,
        grid=(1,),
        in_specs=[pl.BlockSpec((_NW * _KPAD, bpw), lambda i: (0, 0))],
        out_specs=pl.BlockSpec(memory_space=pltpu.SMEM),
        out_shape=jax.ShapeDtypeStruct((1, 1), jnp.float32),
    )(scores.reshape(_NW * _KPAD, bpw))
    return loss[0, 0]


# speedup vs baseline: 5.5259x; 1.0292x over previous
import jax
import jax.numpy as jnp
from jax import lax
from jax.experimental import pallas as pl
from jax.experimental.pallas import tpu as pltpu
from jax.experimental.pallas import tpu_sc as plsc

_VOC = 1000
_EMB = 64
_C = 4
_K = 20
_NIDX = _C + 1 + _K
_NSC = _K + 1
_NW = 32
_L = 16
_GCH = 128
_FK = 12
_LP = (3.507552052950621e-06, 0.9997924357286277, -0.4969779111678143,
       0.31459053537160714, -0.18878267362211323, 0.08172680837613401,
       -0.01720806112146555)


def _tc_gram(u_ref, g_ref):
    U = u_ref[...]
    g_ref[...] = lax.dot_general(
        U, U, (((1,), (1,)), ((), ())),
        preferred_element_type=jnp.float32,
        precision=jax.lax.Precision.HIGHEST)


def _sc_vals(idx_hbm, g_hbm, out_hbm, g_sp, idx_v, lst_v, val_v, s_v, acc_v, sem):
    bpw = idx_v.shape[1]
    cid = lax.axis_index("c")
    sid = lax.axis_index("s")
    w = sid * 2 + cid

    pltpu.sync_copy(idx_hbm.at[w], idx_v)

    @pl.when(sid == 0)
    def _():
        pltpu.sync_copy(g_hbm, g_sp)

    barriered = False
    for c in range(_C):
        def lst_body(g, carry):
            b0 = g * _L
            xc = idx_v[c, pl.ds(b0, _L)]
            for t in range(_NSC):
                tv = idx_v[_C + t, pl.ds(b0, _L)] * _VOC
                lst_v[pl.ds(t * bpw + b0, _L)] = tv + xc
            return carry

        lax.fori_loop(0, bpw // _L, lst_body, 0)

        if not barriered:
            plsc.subcore_barrier()
            barriered = True

        nch = _NSC * bpw // _GCH
        def fire_body(i, carry):
            descs = []
            for k in range(_FK):
                o = (i * _FK + k) * _GCH
                descs.append(pltpu.async_copy(
                    g_sp.at[lst_v.at[pl.ds(o, _GCH)]],
                    val_v.at[pl.ds(o, _GCH)], sem))
            for d in descs:
                d.wait()
            return carry

        lax.fori_loop(0, nch // _FK, fire_body, 0)

        if c < _C - 1:
            def sum_body(g, carry):
                b0 = g * _L
                for t in range(_NSC):
                    v = val_v[pl.ds(t * bpw + b0, _L)]
                    if c == 0:
                        s_v[t, pl.ds(b0, _L)] = v
                    else:
                        s_v[t, pl.ds(b0, _L)] = s_v[t, pl.ds(b0, _L)] + v
                return carry

            lax.fori_loop(0, bpw // _L, sum_body, 0)
        else:
            def loss_body(g, acc):
                b0 = g * _L
                for t in range(_NSC):
                    v = val_v[pl.ds(t * bpw + b0, _L)]
                    sv = (s_v[t, pl.ds(b0, _L)] + v) * (1.0 / _C)
                    wexp = jnp.exp(-jnp.abs(sv))
                    lp = jnp.float32(_LP[6])
                    for a in (_LP[5], _LP[4], _LP[3], _LP[2], _LP[1],
                              _LP[0]):
                        lp = lp * wexp + jnp.float32(a)
                    sp = jnp.maximum(sv, 0.0) + lp
                    if t == 0:
                        acc = acc + (sv - sp)
                    else:
                        acc = acc - sp
                return acc

            acc = lax.fori_loop(0, bpw // _L, loss_body,
                                jnp.zeros((_L,), jnp.float32))
            acc_v[...] = acc

    pltpu.sync_copy(acc_v, out_hbm.at[w])


def kernel(x, target, neg_samples, U):
    B = x.shape[0]
    bpw = B // _NW

    idx_all = jnp.concatenate(
        [x.T, target[None, :], neg_samples.T], axis=0).astype(jnp.int32)
    idx_blk = idx_all.reshape(_NIDX, _NW, bpw).transpose(1, 0, 2)

    gram = pl.pallas_call(
        _tc_gram,
        grid=(1,),
        in_specs=[pl.BlockSpec((_VOC, _EMB), lambda i: (0, 0))],
        out_specs=pl.BlockSpec((_VOC, _VOC), lambda i: (0, 0)),
        out_shape=jax.ShapeDtypeStruct((_VOC, _VOC), jnp.float32),
    )(U)

    mesh = plsc.VectorSubcoreMesh(core_axis_name="c", subcore_axis_name="s")
    partials = pl.kernel(
        _sc_vals,
        out_type=jax.ShapeDtypeStruct((_NW, _L), jnp.float32),
        mesh=mesh,
        scratch_types=[
            pltpu.VMEM_SHARED((_VOC * _VOC,), jnp.float32),
            pltpu.VMEM((_NIDX, bpw), jnp.int32),
            pltpu.VMEM((_NSC * bpw,), jnp.int32),
            pltpu.VMEM((_NSC * bpw,), jnp.float32),
            pltpu.VMEM((_NSC, bpw), jnp.float32),
            pltpu.VMEM((_L,), jnp.float32),
            pltpu.SemaphoreType.DMA,
        ],
        compiler_params=pltpu.CompilerParams(needs_layout_passes=False),
    )(idx_blk, gram.reshape(-1))

    return -jnp.sum(partials)
